# baseline (speedup 1.0000x reference)
import functools

import jax
import jax.numpy as jnp
from jax import lax
from jax.experimental import pallas as pl
from jax.experimental.pallas import tpu as pltpu
from jax.experimental.pallas import tpu_sc as plsc

NC = 2
NS = 16
NW = NC * NS
CH = 128


@functools.partial(jax.jit, static_argnames=())
def _encode(idx, table):
    N = idx.shape[0]
    D = table.shape[1]
    n_per_w = N // NW
    n_ch = n_per_w // CH

    mesh = plsc.VectorSubcoreMesh(core_axis_name="c", subcore_axis_name="s")

    @functools.partial(
        pl.kernel,
        mesh=mesh,
        compiler_params=pltpu.CompilerParams(use_tc_tiling_on_sc=False),
        out_type=jax.ShapeDtypeStruct((N, D), jnp.float32),
        scratch_types=[
            pltpu.VMEM((n_per_w,), jnp.int32),
            pltpu.VMEM((CH, D), jnp.float32),
            pltpu.SemaphoreType.DMA,
        ],
    )
    def enc(table_hbm, idx_hbm, out_hbm, idx_v, rows_v, gsem):
        wid = lax.axis_index("s") * NC + lax.axis_index("c")
        base = wid * n_per_w
        pltpu.sync_copy(idx_hbm.at[pl.ds(base, n_per_w)], idx_v)

        def body(j, carry):
            pltpu.async_copy(
                table_hbm.at[idx_v.at[pl.ds(j * CH, CH)]], rows_v, gsem
            ).wait()
            pltpu.sync_copy(rows_v, out_hbm.at[pl.ds(base + j * CH, CH)])
            return carry

        lax.fori_loop(0, n_ch, body, 0)

    return enc(table, idx)


def kernel(batch_sent_input, embed_weight):
    B, S = batch_sent_input.shape
    D = embed_weight.shape[1]
    idx = batch_sent_input.reshape(B * S).astype(jnp.int32)
    out = _encode(idx, embed_weight)
    return out.reshape(B, S, D)


# speedup vs baseline: 1.1146x; 1.1146x over previous
import functools

import jax
import jax.numpy as jnp
from jax import lax
from jax.experimental import pallas as pl
from jax.experimental.pallas import tpu as pltpu
from jax.experimental.pallas import tpu_sc as plsc

NC = 2
NS = 16
NW = NC * NS
CH = 128
NBUF = 8
LA = 4


def _encode(idx, table):
    N = idx.shape[0]
    D = table.shape[1]
    n_per_w = N // NW
    n_ch = n_per_w // CH

    mesh = plsc.VectorSubcoreMesh(core_axis_name="c", subcore_axis_name="s")

    @functools.partial(
        pl.kernel,
        mesh=mesh,
        compiler_params=pltpu.CompilerParams(use_tc_tiling_on_sc=False),
        out_type=jax.ShapeDtypeStruct((N, D), jnp.float32),
        scratch_types=(
            [
                pltpu.VMEM((n_per_w,), jnp.int32),
                pltpu.VMEM((NBUF, CH, D), jnp.float32),
            ]
            + [pltpu.SemaphoreType.DMA] * (2 * NBUF)
        ),
    )
    def enc(table_hbm, idx_hbm, out_hbm, idx_v, rows_v, *sems):
        gsem = sems[:NBUF]
        osem = sems[NBUF:]
        wid = lax.axis_index("s") * NC + lax.axis_index("c")
        base = wid * n_per_w
        pltpu.sync_copy(idx_hbm.at[pl.ds(base, n_per_w)], idx_v)

        def start_gather(j, b):
            pltpu.async_copy(
                table_hbm.at[idx_v.at[pl.ds(j * CH, CH)]],
                rows_v.at[b],
                gsem[b],
            )

        def wait_gather(j, b):
            pltpu.make_async_copy(
                table_hbm.at[idx_v.at[pl.ds(j * CH, CH)]],
                rows_v.at[b],
                gsem[b],
            ).wait()

        def start_write(j, b):
            pltpu.async_copy(
                rows_v.at[b],
                out_hbm.at[pl.ds(base + j * CH, CH)],
                osem[b],
            )

        def wait_write(j, b):
            pltpu.make_async_copy(
                rows_v.at[b],
                out_hbm.at[pl.ds(base + j * CH, CH)],
                osem[b],
            ).wait()

        def visit(g, b, issue, reuse):
            wait_gather(g, b)
            start_write(g, b)
            if issue:
                jj = g + LA
                bb = (b + LA) % NBUF
                if reuse:
                    wait_write(jj - NBUF, bb)
                start_gather(jj, bb)

        for g in range(LA):
            start_gather(g, g % NBUF)

        for g in range(NBUF):
            visit(g, g % NBUF, g + LA < n_ch, g + LA >= NBUF)

        def block(blk, carry):
            for b in range(NBUF):
                g = blk * NBUF + b
                visit(g, b, True, True)
            return carry

        lax.fori_loop(1, n_ch // NBUF - 1, block, 0)

        for g in range(n_ch - NBUF, n_ch):
            visit(g, g % NBUF, g + LA < n_ch, True)

        for g in range(n_ch - NBUF, n_ch):
            wait_write(g, g % NBUF)

    return enc(table, idx)


def kernel(batch_sent_input, embed_weight):
    B, S = batch_sent_input.shape
    D = embed_weight.shape[1]
    idx = batch_sent_input.reshape(B * S).astype(jnp.int32)
    out = _encode(idx, embed_weight)
    return out.reshape(B, S, D)
